# baseline (speedup 1.0000x reference)
import functools

import jax
import jax.numpy as jnp
import numpy as np
from jax import lax
from jax.experimental import pallas as pl
from jax.experimental.pallas import tpu as pltpu
from jax.experimental.pallas import tpu_sc as plsc

B, T, FT, FP, V, E = 4096, 50, 8, 8, 1000, 16
ROW = 1 + FP + FP + FT + FT + 1 + 1 + E
VP = 1024
NC, NS, L = 2, 16, 16
NW = NC * NS
EX_W = B // NW
COL_W = EX_W * T

def _sc_body(gsp_hbm, az_hbm, el_hbm, tf_hbm, y_hbm, x_hbm,
             t0_hbm, ids_hbm, tab_hbm, inc_hbm,
             out_hbm,
             gspv, azv, elv, tfv, yv, xv, t0v, idsv, tabv, embl,
             colb0, colb1, incb,
             sem_az, sem_el, sem_gsp, sem_y, sem_x, sem_tf, sem_t0, sem_tab,
             sem_out0, sem_out1):
    wid = lax.axis_index("s") * NC + lax.axis_index("c")
    b0 = wid * EX_W
    lanes = lax.iota(jnp.int32, L)
    colbs = (colb0, colb1)
    sems = (sem_out0, sem_out1)

    cps = [
        pltpu.async_copy(az_hbm.at[:, pl.ds(b0, EX_W)], azv, sem_az),
        pltpu.async_copy(el_hbm.at[:, pl.ds(b0, EX_W)], elv, sem_el),
        pltpu.async_copy(gsp_hbm.at[:, pl.ds(b0, EX_W)], gspv, sem_gsp),
        pltpu.async_copy(inc_hbm, incb, sem_gsp),
        pltpu.async_copy(y_hbm.at[:, pl.ds(b0, EX_W)], yv, sem_y),
        pltpu.async_copy(x_hbm.at[:, pl.ds(b0, EX_W)], xv, sem_x),
        pltpu.async_copy(tf_hbm.at[:, :, pl.ds(b0, EX_W)], tfv, sem_tf),
        pltpu.async_copy(t0_hbm.at[:, pl.ds(b0, EX_W)], t0v, sem_t0),
        pltpu.async_copy(ids_hbm.at[pl.ds(b0, EX_W)], idsv, sem_tab),
        pltpu.async_copy(tab_hbm, tabv, sem_tab),
    ]

    def build_embl():
        def embl_c(c, _):
            @plsc.parallel_loop(0, EX_W // L, unroll=4)
            def _(g):
                ev = lanes + g * L
                idv = plsc.load_gather(idsv, [ev])
                idv = jnp.minimum(jnp.maximum(idv, 0), V - 1)
                row = plsc.load_gather(
                    tabv, [jnp.full((L,), c, jnp.int32), idv])
                embl[pl.ds(c * EX_W + g * L, L)] = row
            return 0

        lax.fori_loop(0, E, embl_c, 0)

    def drain(p):
        pltpu.make_async_copy(
            out_hbm.at[pl.ds(0, COL_W)], colbs[p], sems[p]).wait()

    def emit(colb, p, c):
        pltpu.async_copy(
            colb, out_hbm.at[pl.ds(c * (B * T) + wid * COL_W, COL_W)],
            sems[p])

    GPR = EX_W // L

    def fill_transpose(colb, src, marker):
        @plsc.parallel_loop(0, T * GPR, unroll=8)
        def _(i):
            t = i >> 3
            ev = lanes + (i & (GPR - 1)) * L
            v = plsc.load_gather(src, [jnp.full((L,), t, jnp.int32), ev])
            if marker:
                v = 1.0 + incv * v
            plsc.store_scatter(colb, [ev * T + t], v)

    def fill_bcast(colb, src2, j, flat_base=None):
        @plsc.parallel_loop(0, EX_W // L, unroll=8)
        def _(g):
            if flat_base is None:
                vv = plsc.load_gather(
                    src2, [jnp.full((L,), j, jnp.int32), lanes + g * L])
            else:
                vv = plsc.load_gather(src2, [flat_base + lanes + g * L])
            for l in range(L):
                sp = jnp.full((L,), vv[l], jnp.float32)
                base = (g * L + l) * T
                colb[pl.ds(base, L)] = sp
                colb[pl.ds(base + L, L)] = sp
                colb[pl.ds(base + 2 * L, L)] = sp
                colb[pl.ds(base + T - L, L)] = sp

    def fill_tf(colb, j):
        jv = jnp.full((L,), j, jnp.int32)

        @plsc.parallel_loop(0, T * GPR, unroll=8)
        def _(i):
            t = i >> 3
            ev = lanes + (i & (GPR - 1)) * L
            v = plsc.load_gather(tfv, [jnp.full((L,), t, jnp.int32), jv, ev])
            plsc.store_scatter(colb, [ev * T + t], v)

    cps[0].wait()
    fill_transpose(colb0, azv, False)
    emit(colb0, 0, 33)
    cps[1].wait()
    fill_transpose(colb1, elv, False)
    emit(colb1, 1, 34)

    cps[2].wait()
    cps[3].wait()
    incv = incb[...]
    drain(0)
    fill_transpose(colb0, gspv, True)
    emit(colb0, 0, 0)
    cps[4].wait()

    def pair_group(src2, col0, npair, off):
        def body(jj, _):
            j0 = off + 2 * jj
            c0 = col0 + 2 * jj
            drain(1)
            fill_bcast(colb1, src2, j0)
            emit(colb1, 1, c0)
            drain(0)
            fill_bcast(colb0, src2, j0 + 1)
            emit(colb0, 0, c0 + 1)
            return 0
        lax.fori_loop(0, npair, body, 0)

    pair_group(yv, 1, FP // 2, 0)
    cps[5].wait()
    pair_group(xv, 9, FP // 2, 0)
    cps[6].wait()

    def tf_pairs(jj, _):
        j0 = 2 * jj
        drain(1)
        fill_tf(colb1, j0)
        emit(colb1, 1, 17 + j0)
        drain(0)
        fill_tf(colb0, j0 + 1)
        emit(colb0, 0, 18 + j0)
        return 0

    lax.fori_loop(0, FT // 2, tf_pairs, 0)

    cps[7].wait()
    pair_group(t0v, 25, FT // 2, 0)

    cps[8].wait()
    cps[9].wait()
    build_embl()

    def emb_pairs(jj, _):
        j0 = 2 * jj
        drain(1)
        fill_bcast(colb1, embl, None, flat_base=j0 * EX_W)
        emit(colb1, 1, 35 + j0)
        drain(0)
        fill_bcast(colb0, embl, None, flat_base=(j0 + 1) * EX_W)
        emit(colb0, 0, 36 + j0)
        return 0

    lax.fori_loop(0, E // 2, emb_pairs, 0)

    drain(0)
    drain(1)


@functools.cache
def _get_sc_kernel():
    return pl.kernel(
        _sc_body,
        out_type=jax.ShapeDtypeStruct((B * T * ROW,), jnp.float32),
        mesh=plsc.VectorSubcoreMesh(core_axis_name="c", subcore_axis_name="s"),
        compiler_params=pltpu.CompilerParams(needs_layout_passes=False),
        scratch_types=[
            pltpu.VMEM((T, EX_W), jnp.float32),
            pltpu.VMEM((T, EX_W), jnp.float32),
            pltpu.VMEM((T, EX_W), jnp.float32),
            pltpu.VMEM((T, FT, EX_W), jnp.float32),
            pltpu.VMEM((FP, EX_W), jnp.float32),
            pltpu.VMEM((FP, EX_W), jnp.float32),
            pltpu.VMEM((FT, EX_W), jnp.float32),
            pltpu.VMEM((EX_W,), jnp.int32),
            pltpu.VMEM((E, VP), jnp.float32),
            pltpu.VMEM((E * EX_W,), jnp.float32),
            pltpu.VMEM((COL_W,), jnp.float32),
            pltpu.VMEM((COL_W,), jnp.float32),
            pltpu.VMEM((L,), jnp.float32),
        ] + [pltpu.SemaphoreType.DMA] * 10,
    )


def kernel(gsp, gsp_solar_azimuth, gsp_solar_elevation, gsp_time_utc_fourier,
           gsp_time_utc_fourier_t0, gsp_y_osgb_fourier, gsp_x_osgb_fourier,
           gsp_id, emb_table, gsp_t0_idx, include_history):
    del gsp_t0_idx
    ids = gsp_id.reshape(B).astype(jnp.int32)
    inc = jnp.full((L,), jnp.asarray(include_history, jnp.float32))
    tab = jnp.pad(emb_table.T, ((0, 0), (0, VP - V)))
    out = _get_sc_kernel()(
        gsp.T,
        gsp_solar_azimuth.T,
        gsp_solar_elevation.T,
        gsp_time_utc_fourier.transpose(1, 2, 0),
        gsp_y_osgb_fourier.reshape(B, FP).T,
        gsp_x_osgb_fourier.reshape(B, FP).T,
        gsp_time_utc_fourier_t0.T,
        ids,
        tab,
        inc,
    )
    return out.reshape(ROW, 1, B * T).transpose(2, 1, 0)


# speedup vs baseline: 1.0720x; 1.0720x over previous
import functools

import jax
import jax.numpy as jnp
import numpy as np
from jax import lax
from jax.experimental import pallas as pl
from jax.experimental.pallas import tpu as pltpu
from jax.experimental.pallas import tpu_sc as plsc

B, T, FT, FP, V, E = 4096, 50, 8, 8, 1000, 16
ROW = 1 + FP + FP + FT + FT + 1 + 1 + E
VP = 1024
NC, NS, L = 2, 16, 16
NW = NC * NS
EX_W = B // NW
COL_W = EX_W * T

def _sc_body(gsp_hbm, az_hbm, el_hbm, tf_hbm, y_hbm, x_hbm,
             t0_hbm, ids_hbm, tab_hbm, inc_hbm,
             out_hbm,
             gspv, azv, elv, tfv, yv, xv, t0v, idsv, tabv, embl,
             colb0, colb1, incb,
             sem_az, sem_el, sem_gsp, sem_y, sem_x, sem_tf, sem_t0, sem_tab,
             sem_out0, sem_out1):
    wid = lax.axis_index("s") * NC + lax.axis_index("c")
    b0 = wid * EX_W
    lanes = lax.iota(jnp.int32, L)
    colbs = (colb0, colb1)
    sems = (sem_out0, sem_out1)

    cps = [
        pltpu.async_copy(az_hbm.at[:, pl.ds(b0, EX_W)], azv, sem_az),
        pltpu.async_copy(el_hbm.at[:, pl.ds(b0, EX_W)], elv, sem_el),
        pltpu.async_copy(gsp_hbm.at[:, pl.ds(b0, EX_W)], gspv, sem_gsp),
        pltpu.async_copy(inc_hbm, incb, sem_gsp),
        pltpu.async_copy(y_hbm.at[:, pl.ds(b0, EX_W)], yv, sem_y),
        pltpu.async_copy(x_hbm.at[:, pl.ds(b0, EX_W)], xv, sem_x),
        pltpu.async_copy(tf_hbm.at[:, :, pl.ds(b0, EX_W)], tfv, sem_tf),
        pltpu.async_copy(t0_hbm.at[:, pl.ds(b0, EX_W)], t0v, sem_t0),
        pltpu.async_copy(ids_hbm.at[pl.ds(b0, EX_W)], idsv, sem_tab),
        pltpu.async_copy(tab_hbm, tabv, sem_tab),
    ]

    def build_embl():
        def embl_c(c, _):
            @plsc.parallel_loop(0, EX_W // L, unroll=4)
            def _(g):
                ev = lanes + g * L
                idv = plsc.load_gather(idsv, [ev])
                idv = jnp.minimum(jnp.maximum(idv, 0), V - 1)
                row = plsc.load_gather(
                    tabv, [jnp.full((L,), c, jnp.int32), idv])
                embl[pl.ds(c * EX_W + g * L, L)] = row
            return 0

        lax.fori_loop(0, E, embl_c, 0)

    def drain(p):
        pltpu.make_async_copy(
            out_hbm.at[pl.ds(0, COL_W)], colbs[p], sems[p]).wait()

    def emit(colb, p, c):
        pltpu.async_copy(
            colb, out_hbm.at[pl.ds(c * (B * T) + wid * COL_W, COL_W)],
            sems[p])

    GPR = EX_W // L

    def fill_transpose(colb, src, marker):
        @plsc.parallel_loop(0, T * GPR, unroll=8)
        def _(i):
            t = i >> 3
            ev = lanes + (i & (GPR - 1)) * L
            v = plsc.load_gather(src, [jnp.full((L,), t, jnp.int32), ev])
            if marker:
                v = 1.0 + incv * v
            plsc.store_scatter(colb, [ev * T + t], v)

    def fill_bcast(colb, src2, j, flat_base=None):
        @plsc.parallel_loop(0, EX_W // L, unroll=4)
        def _(g):
            if flat_base is None:
                vv = plsc.load_gather(
                    src2, [jnp.full((L,), j, jnp.int32), lanes + g * L])
            else:
                vv = plsc.load_gather(src2, [flat_base + lanes + g * L])
            for l in range(L):
                sp = jnp.full((L,), vv[l], jnp.float32)
                base = (g * L + l) * T
                colb[pl.ds(base, L)] = sp
                colb[pl.ds(base + L, L)] = sp
                colb[pl.ds(base + 2 * L, L)] = sp
                colb[pl.ds(base + T - L, L)] = sp

    def fill_tf(colb, j):
        jv = jnp.full((L,), j, jnp.int32)

        @plsc.parallel_loop(0, T * GPR, unroll=8)
        def _(i):
            t = i >> 3
            ev = lanes + (i & (GPR - 1)) * L
            v = plsc.load_gather(tfv, [jnp.full((L,), t, jnp.int32), jv, ev])
            plsc.store_scatter(colb, [ev * T + t], v)

    cps[0].wait()
    fill_transpose(colb0, azv, False)
    emit(colb0, 0, 33)
    cps[1].wait()
    fill_transpose(colb1, elv, False)
    emit(colb1, 1, 34)

    cps[2].wait()
    cps[3].wait()
    incv = incb[...]
    drain(0)
    fill_transpose(colb0, gspv, True)
    emit(colb0, 0, 0)
    cps[4].wait()

    def pair_group(src2, col0, npair, off):
        def body(jj, _):
            j0 = off + 2 * jj
            c0 = col0 + 2 * jj
            drain(1)
            fill_bcast(colb1, src2, j0)
            emit(colb1, 1, c0)
            drain(0)
            fill_bcast(colb0, src2, j0 + 1)
            emit(colb0, 0, c0 + 1)
            return 0
        lax.fori_loop(0, npair, body, 0)

    pair_group(yv, 1, FP // 2, 0)
    cps[5].wait()
    pair_group(xv, 9, FP // 2, 0)
    cps[6].wait()

    def tf_pairs(jj, _):
        j0 = 2 * jj
        drain(1)
        fill_tf(colb1, j0)
        emit(colb1, 1, 17 + j0)
        drain(0)
        fill_tf(colb0, j0 + 1)
        emit(colb0, 0, 18 + j0)
        return 0

    lax.fori_loop(0, FT // 2, tf_pairs, 0)

    cps[7].wait()
    pair_group(t0v, 25, FT // 2, 0)

    cps[8].wait()
    cps[9].wait()
    build_embl()

    def emb_pairs(jj, _):
        j0 = 2 * jj
        drain(1)
        fill_bcast(colb1, embl, None, flat_base=j0 * EX_W)
        emit(colb1, 1, 35 + j0)
        drain(0)
        fill_bcast(colb0, embl, None, flat_base=(j0 + 1) * EX_W)
        emit(colb0, 0, 36 + j0)
        return 0

    lax.fori_loop(0, E // 2, emb_pairs, 0)

    drain(0)
    drain(1)


@functools.cache
def _get_sc_kernel():
    return pl.kernel(
        _sc_body,
        out_type=jax.ShapeDtypeStruct((B * T * ROW,), jnp.float32),
        mesh=plsc.VectorSubcoreMesh(core_axis_name="c", subcore_axis_name="s"),
        compiler_params=pltpu.CompilerParams(needs_layout_passes=False),
        scratch_types=[
            pltpu.VMEM((T, EX_W), jnp.float32),
            pltpu.VMEM((T, EX_W), jnp.float32),
            pltpu.VMEM((T, EX_W), jnp.float32),
            pltpu.VMEM((T, FT, EX_W), jnp.float32),
            pltpu.VMEM((FP, EX_W), jnp.float32),
            pltpu.VMEM((FP, EX_W), jnp.float32),
            pltpu.VMEM((FT, EX_W), jnp.float32),
            pltpu.VMEM((EX_W,), jnp.int32),
            pltpu.VMEM((E, VP), jnp.float32),
            pltpu.VMEM((E * EX_W,), jnp.float32),
            pltpu.VMEM((COL_W,), jnp.float32),
            pltpu.VMEM((COL_W,), jnp.float32),
            pltpu.VMEM((L,), jnp.float32),
        ] + [pltpu.SemaphoreType.DMA] * 10,
    )


def kernel(gsp, gsp_solar_azimuth, gsp_solar_elevation, gsp_time_utc_fourier,
           gsp_time_utc_fourier_t0, gsp_y_osgb_fourier, gsp_x_osgb_fourier,
           gsp_id, emb_table, gsp_t0_idx, include_history):
    del gsp_t0_idx
    ids = gsp_id.reshape(B).astype(jnp.int32)
    inc = jnp.full((L,), jnp.asarray(include_history, jnp.float32))
    tab = jnp.pad(emb_table.T, ((0, 0), (0, VP - V)))
    out = _get_sc_kernel()(
        gsp.T,
        gsp_solar_azimuth.T,
        gsp_solar_elevation.T,
        gsp_time_utc_fourier.transpose(1, 2, 0),
        gsp_y_osgb_fourier.reshape(B, FP).T,
        gsp_x_osgb_fourier.reshape(B, FP).T,
        gsp_time_utc_fourier_t0.T,
        ids,
        tab,
        inc,
    )
    return out.reshape(ROW, 1, B * T).transpose(2, 1, 0)


# speedup vs baseline: 1.1183x; 1.0432x over previous
import functools

import jax
import jax.numpy as jnp
import numpy as np
from jax import lax
from jax.experimental import pallas as pl
from jax.experimental.pallas import tpu as pltpu
from jax.experimental.pallas import tpu_sc as plsc

B, T, FT, FP, V, E = 4096, 50, 8, 8, 1000, 16
ROW = 1 + FP + FP + FT + FT + 1 + 1 + E
VP = 1024
NC, NS, L = 2, 16, 16
NW = NC * NS
EX_W = B // NW
COL_W = EX_W * T

def _sc_body(gsp_hbm, az_hbm, el_hbm, tf_hbm, y_hbm, x_hbm,
             t0_hbm, ids_hbm, tab_hbm, inc_hbm,
             out_hbm,
             gspv, azv, elv, tfv, yv, xv, t0v, idsv, tabv, embl,
             colb0, colb1, incb,
             sem_az, sem_el, sem_gsp, sem_y, sem_x, sem_tf, sem_t0, sem_tab,
             sem_out0, sem_out1):
    wid = lax.axis_index("s") * NC + lax.axis_index("c")
    b0 = wid * EX_W
    lanes = lax.iota(jnp.int32, L)
    colbs = (colb0, colb1)
    sems = (sem_out0, sem_out1)

    cps = [
        pltpu.async_copy(az_hbm.at[:, pl.ds(b0, EX_W)], azv, sem_az),
        pltpu.async_copy(el_hbm.at[:, pl.ds(b0, EX_W)], elv, sem_el),
        pltpu.async_copy(gsp_hbm.at[:, pl.ds(b0, EX_W)], gspv, sem_gsp),
        pltpu.async_copy(inc_hbm, incb, sem_gsp),
        pltpu.async_copy(y_hbm.at[:, pl.ds(b0, EX_W)], yv, sem_y),
        pltpu.async_copy(x_hbm.at[:, pl.ds(b0, EX_W)], xv, sem_x),
        pltpu.async_copy(tf_hbm.at[:, :, pl.ds(b0, EX_W)], tfv, sem_tf),
        pltpu.async_copy(t0_hbm.at[:, pl.ds(b0, EX_W)], t0v, sem_t0),
        pltpu.async_copy(ids_hbm.at[pl.ds(b0, EX_W)], idsv, sem_tab),
        pltpu.async_copy(tab_hbm, tabv, sem_tab),
    ]

    def build_embl():
        def embl_c(c, _):
            @plsc.parallel_loop(0, EX_W // L, unroll=4)
            def _(g):
                ev = lanes + g * L
                idv = plsc.load_gather(idsv, [ev])
                idv = jnp.minimum(jnp.maximum(idv, 0), V - 1)
                row = plsc.load_gather(
                    tabv, [jnp.full((L,), c, jnp.int32), idv])
                embl[pl.ds(c * EX_W + g * L, L)] = row
            return 0

        lax.fori_loop(0, E, embl_c, 0)

    def drain(p):
        pltpu.make_async_copy(
            out_hbm.at[pl.ds(0, COL_W)], colbs[p], sems[p]).wait()

    def emit(colb, p, c):
        pltpu.async_copy(
            colb, out_hbm.at[pl.ds(c * (B * T) + wid * COL_W, COL_W)],
            sems[p])

    GPR = EX_W // L

    def fill_transpose(colb, src, marker):
        @plsc.parallel_loop(0, T * GPR, unroll=8)
        def _(i):
            t = i >> 3
            ev = lanes + (i & (GPR - 1)) * L
            v = plsc.load_gather(src, [jnp.full((L,), t, jnp.int32), ev])
            if marker:
                v = 1.0 + incv * v
            plsc.store_scatter(colb, [ev * T + t], v)

    lo2 = lanes < 2

    def fill_bcast(colb, src2, j, flat_base=None):
        @plsc.parallel_loop(0, EX_W // L, unroll=4)
        def _(g):
            if flat_base is None:
                vv = plsc.load_gather(
                    src2, [jnp.full((L,), j, jnp.int32), lanes + g * L])
            else:
                vv = plsc.load_gather(src2, [flat_base + lanes + g * L])
            for l in range(0, L, 2):
                sp0 = jnp.full((L,), vv[l], jnp.float32)
                sp1 = jnp.full((L,), vv[l + 1], jnp.float32)
                mix = jnp.where(lo2, sp0, sp1)
                base = (g * L + l) * T
                colb[pl.ds(base, L)] = sp0
                colb[pl.ds(base + L, L)] = sp0
                colb[pl.ds(base + 2 * L, L)] = sp0
                colb[pl.ds(base + 3 * L, L)] = mix
                colb[pl.ds(base + 4 * L, L)] = sp1
                colb[pl.ds(base + 5 * L, L)] = sp1
                colb[pl.ds(base + 2 * T - L, L)] = sp1

    def fill_tf(colb, j):
        jv = jnp.full((L,), j, jnp.int32)

        @plsc.parallel_loop(0, T * GPR, unroll=8)
        def _(i):
            t = i >> 3
            ev = lanes + (i & (GPR - 1)) * L
            v = plsc.load_gather(tfv, [jnp.full((L,), t, jnp.int32), jv, ev])
            plsc.store_scatter(colb, [ev * T + t], v)

    cps[0].wait()
    fill_transpose(colb0, azv, False)
    emit(colb0, 0, 33)
    cps[1].wait()
    fill_transpose(colb1, elv, False)
    emit(colb1, 1, 34)

    cps[2].wait()
    cps[3].wait()
    incv = incb[...]
    drain(0)
    fill_transpose(colb0, gspv, True)
    emit(colb0, 0, 0)
    cps[4].wait()

    def pair_group(src2, col0, npair, off):
        def body(jj, _):
            j0 = off + 2 * jj
            c0 = col0 + 2 * jj
            drain(1)
            fill_bcast(colb1, src2, j0)
            emit(colb1, 1, c0)
            drain(0)
            fill_bcast(colb0, src2, j0 + 1)
            emit(colb0, 0, c0 + 1)
            return 0
        lax.fori_loop(0, npair, body, 0)

    pair_group(yv, 1, FP // 2, 0)
    cps[5].wait()
    pair_group(xv, 9, FP // 2, 0)
    cps[6].wait()

    def tf_pairs(jj, _):
        j0 = 2 * jj
        drain(1)
        fill_tf(colb1, j0)
        emit(colb1, 1, 17 + j0)
        drain(0)
        fill_tf(colb0, j0 + 1)
        emit(colb0, 0, 18 + j0)
        return 0

    lax.fori_loop(0, FT // 2, tf_pairs, 0)

    cps[7].wait()
    pair_group(t0v, 25, FT // 2, 0)

    cps[8].wait()
    cps[9].wait()
    build_embl()

    def emb_pairs(jj, _):
        j0 = 2 * jj
        drain(1)
        fill_bcast(colb1, embl, None, flat_base=j0 * EX_W)
        emit(colb1, 1, 35 + j0)
        drain(0)
        fill_bcast(colb0, embl, None, flat_base=(j0 + 1) * EX_W)
        emit(colb0, 0, 36 + j0)
        return 0

    lax.fori_loop(0, E // 2, emb_pairs, 0)

    drain(0)
    drain(1)


@functools.cache
def _get_sc_kernel():
    return pl.kernel(
        _sc_body,
        out_type=jax.ShapeDtypeStruct((B * T * ROW,), jnp.float32),
        mesh=plsc.VectorSubcoreMesh(core_axis_name="c", subcore_axis_name="s"),
        compiler_params=pltpu.CompilerParams(needs_layout_passes=False),
        scratch_types=[
            pltpu.VMEM((T, EX_W), jnp.float32),
            pltpu.VMEM((T, EX_W), jnp.float32),
            pltpu.VMEM((T, EX_W), jnp.float32),
            pltpu.VMEM((T, FT, EX_W), jnp.float32),
            pltpu.VMEM((FP, EX_W), jnp.float32),
            pltpu.VMEM((FP, EX_W), jnp.float32),
            pltpu.VMEM((FT, EX_W), jnp.float32),
            pltpu.VMEM((EX_W,), jnp.int32),
            pltpu.VMEM((E, VP), jnp.float32),
            pltpu.VMEM((E * EX_W,), jnp.float32),
            pltpu.VMEM((COL_W,), jnp.float32),
            pltpu.VMEM((COL_W,), jnp.float32),
            pltpu.VMEM((L,), jnp.float32),
        ] + [pltpu.SemaphoreType.DMA] * 10,
    )


def kernel(gsp, gsp_solar_azimuth, gsp_solar_elevation, gsp_time_utc_fourier,
           gsp_time_utc_fourier_t0, gsp_y_osgb_fourier, gsp_x_osgb_fourier,
           gsp_id, emb_table, gsp_t0_idx, include_history):
    del gsp_t0_idx
    ids = gsp_id.reshape(B).astype(jnp.int32)
    inc = jnp.full((L,), jnp.asarray(include_history, jnp.float32))
    tab = jnp.pad(emb_table.T, ((0, 0), (0, VP - V)))
    out = _get_sc_kernel()(
        gsp.T,
        gsp_solar_azimuth.T,
        gsp_solar_elevation.T,
        gsp_time_utc_fourier.transpose(1, 2, 0),
        gsp_y_osgb_fourier.reshape(B, FP).T,
        gsp_x_osgb_fourier.reshape(B, FP).T,
        gsp_time_utc_fourier_t0.T,
        ids,
        tab,
        inc,
    )
    return out.reshape(ROW, 1, B * T).transpose(2, 1, 0)
